# initial kernel scaffold (unmeasured)
import jax
import jax.numpy as jnp
from jax import lax
from jax.experimental import pallas as pl
from jax.experimental.pallas import tpu as pltpu

N_DEV = 8
SQ = 256
D = 1024
DH = 128
HQ_PER = 8
KV_COLS = 256
SCALE = 0.08838834764831843


def kernel(x, Wq, Wo, Wk, Wv):
    i = lax.axis_index("i")
    Wk_s = lax.dynamic_slice(Wk, (0, i * KV_COLS), (D, KV_COLS))
    Wv_s = lax.dynamic_slice(Wv, (0, i * KV_COLS), (D, KV_COLS))

    def body(x_ref, wq_ref, wo_ref, wk_ref, wv_ref, out_ref,
             comm_ref, acc_ref, send_sems, recv_sems):
        my = lax.axis_index("i")
        left = lax.rem(my + N_DEV - 1, N_DEV)
        right = lax.rem(my + 1, N_DEV)

        barrier = pltpu.get_barrier_semaphore()
        for nbr in (left, right):
            pl.semaphore_signal(
                barrier, inc=1,
                device_id=(nbr,), device_id_type=pl.DeviceIdType.MESH,
            )
        pl.semaphore_wait(barrier, 2)

        xv = x_ref[0, :, :]
        q = jnp.dot(xv, wq_ref[...], preferred_element_type=jnp.float32)
        k = jnp.dot(xv, wk_ref[...], preferred_element_type=jnp.float32)
        v = jnp.dot(xv, wv_ref[...], preferred_element_type=jnp.float32)

        outs = []
        for h in range(HQ_PER):
            qh = q[:, h * DH:(h + 1) * DH]
            g = h // 4
            kh = k[:, g * DH:(g + 1) * DH]
            vh = v[:, g * DH:(g + 1) * DH]
            s = lax.dot_general(
                qh, kh, (((1,), (1,)), ((), ())),
                preferred_element_type=jnp.float32,
            ) * SCALE
            m = jnp.max(s, axis=-1, keepdims=True)
            p = jnp.exp(s - m)
            l = jnp.sum(p, axis=-1, keepdims=True)
            oh = jnp.dot(p, vh, preferred_element_type=jnp.float32) / l
            outs.append(oh)
        o = jnp.concatenate(outs, axis=1)
        partial = jnp.dot(o, wo_ref[...], preferred_element_type=jnp.float32)

        acc_ref[...] = partial
        comm_ref[0] = partial

        for h in range(N_DEV - 1):
            rdma = pltpu.make_async_remote_copy(
                src_ref=comm_ref.at[h],
                dst_ref=comm_ref.at[h + 1],
                send_sem=send_sems.at[h],
                recv_sem=recv_sems.at[h + 1],
                device_id=(right,),
                device_id_type=pl.DeviceIdType.MESH,
            )
            rdma.start()
            rdma.wait()
            acc_ref[...] += comm_ref[h + 1]

        out_ref[0, :, :] = acc_ref[...]

    return pl.pallas_call(
        body,
        out_shape=jax.ShapeDtypeStruct((1, SQ, D), jnp.float32),
        in_specs=[pl.BlockSpec(memory_space=pltpu.VMEM)] * 5,
        out_specs=pl.BlockSpec(memory_space=pltpu.VMEM),
        scratch_shapes=[
            pltpu.VMEM((N_DEV, SQ, D), jnp.float32),
            pltpu.VMEM((SQ, D), jnp.float32),
            pltpu.SemaphoreType.DMA((N_DEV,)),
            pltpu.SemaphoreType.DMA((N_DEV,)),
        ],
        compiler_params=pltpu.CompilerParams(collective_id=0),
    )(x, Wq, Wo, Wk, Wv)


# baseline (device time: 107516 ns/iter reference)
import jax
import jax.numpy as jnp
from jax import lax
from jax.experimental import pallas as pl
from jax.experimental.pallas import tpu as pltpu

N_DEV = 8
SQ = 256
D = 1024
DH = 128
HQ_PER = 8
KV_COLS = 256
SCALE = 0.08838834764831843


def kernel(x, Wq, Wo, Wk, Wv):
    i = lax.axis_index("i")
    Wk_s = lax.dynamic_slice(Wk, (0, i * KV_COLS), (D, KV_COLS))
    Wv_s = lax.dynamic_slice(Wv, (0, i * KV_COLS), (D, KV_COLS))

    def body(x_ref, wq_ref, wo_ref, wk_ref, wv_ref, out_ref,
             comm_ref, acc_ref, send_sems, recv_sems):
        my = lax.axis_index("i")
        left = lax.rem(my + N_DEV - 1, N_DEV)
        right = lax.rem(my + 1, N_DEV)

        barrier = pltpu.get_barrier_semaphore()
        for nbr in (left, right):
            pl.semaphore_signal(
                barrier, inc=1,
                device_id=(nbr,), device_id_type=pl.DeviceIdType.MESH,
            )
        pl.semaphore_wait(barrier, 2)

        xv = x_ref[0, :, :]
        q = jnp.dot(xv, wq_ref[...], preferred_element_type=jnp.float32)
        k = jnp.dot(xv, wk_ref[...], preferred_element_type=jnp.float32)
        v = jnp.dot(xv, wv_ref[...], preferred_element_type=jnp.float32)

        outs = []
        for h in range(HQ_PER):
            qh = q[:, h * DH:(h + 1) * DH]
            g = h // 4
            kh = k[:, g * DH:(g + 1) * DH]
            vh = v[:, g * DH:(g + 1) * DH]
            s = lax.dot_general(
                qh, kh, (((1,), (1,)), ((), ())),
                preferred_element_type=jnp.float32,
            ) * SCALE
            m = jnp.max(s, axis=-1, keepdims=True)
            p = jnp.exp(s - m)
            l = jnp.sum(p, axis=-1, keepdims=True)
            oh = jnp.dot(p, vh, preferred_element_type=jnp.float32) / l
            outs.append(oh)
        o = jnp.concatenate(outs, axis=1)
        partial = jnp.dot(o, wo_ref[...], preferred_element_type=jnp.float32)

        acc_ref[...] = partial
        comm_ref[0] = partial

        for h in range(N_DEV - 1):
            rdma = pltpu.make_async_remote_copy(
                src_ref=comm_ref.at[h],
                dst_ref=comm_ref.at[h + 1],
                send_sem=send_sems.at[h],
                recv_sem=recv_sems.at[h + 1],
                device_id=(right,),
                device_id_type=pl.DeviceIdType.MESH,
            )
            rdma.start()
            rdma.wait()
            acc_ref[...] += comm_ref[h + 1]

        out_ref[0, :, :] = acc_ref[...]

    return pl.pallas_call(
        body,
        out_shape=jax.ShapeDtypeStruct((1, SQ, D), jnp.float32),
        in_specs=[pl.BlockSpec(memory_space=pltpu.VMEM)] * 5,
        out_specs=pl.BlockSpec(memory_space=pltpu.VMEM),
        scratch_shapes=[
            pltpu.VMEM((N_DEV, SQ, D), jnp.float32),
            pltpu.VMEM((SQ, D), jnp.float32),
            pltpu.SemaphoreType.DMA((N_DEV,)),
            pltpu.SemaphoreType.DMA((N_DEV,)),
        ],
        compiler_params=pltpu.CompilerParams(collective_id=0),
    )(x, Wq, Wo, Wk_s, Wv_s)


# device time: 46664 ns/iter; 2.3040x vs baseline; 2.3040x over previous
import jax
import jax.numpy as jnp
from jax import lax
from jax.experimental import pallas as pl
from jax.experimental.pallas import tpu as pltpu

N_DEV = 8
SQ = 256
D = 1024
DH = 128
HQ_PER = 8
KV_COLS = 256
SCALE = 0.08838834764831843


def kernel(x, Wq, Wo, Wk, Wv):
    i = lax.axis_index("i")
    Wk_s = lax.dynamic_slice(Wk, (0, i * KV_COLS), (D, KV_COLS))
    Wv_s = lax.dynamic_slice(Wv, (0, i * KV_COLS), (D, KV_COLS))

    def body(x_ref, wq_ref, wo_ref, wk_ref, wv_ref, out_ref,
             acc_ref, rbuf0, rbuf1, rbuf2, send_sems, recv_sems):
        my = lax.axis_index("i")
        b2 = (my >> 2) & 1
        b1 = (my >> 1) & 1
        b0 = my & 1
        partners = [my ^ 4, my ^ 2, my ^ 1]

        barrier = pltpu.get_barrier_semaphore()
        for p in partners:
            pl.semaphore_signal(
                barrier, inc=1,
                device_id=(p,), device_id_type=pl.DeviceIdType.MESH,
            )
        pl.semaphore_wait(barrier, 3)

        xv = x_ref[0, :, :]
        q = jnp.dot(xv, wq_ref[...], preferred_element_type=jnp.float32)
        k = jnp.dot(xv, wk_ref[...], preferred_element_type=jnp.float32)
        v = jnp.dot(xv, wv_ref[...], preferred_element_type=jnp.float32)

        outs = []
        for h in range(HQ_PER):
            qh = q[:, h * DH:(h + 1) * DH]
            g = h // 4
            kh = k[:, g * DH:(g + 1) * DH]
            vh = v[:, g * DH:(g + 1) * DH]
            s = lax.dot_general(
                qh, kh, (((1,), (1,)), ((), ())),
                preferred_element_type=jnp.float32,
            ) * SCALE
            m = jnp.max(s, axis=-1, keepdims=True)
            p = jnp.exp(s - m)
            l = jnp.sum(p, axis=-1, keepdims=True)
            oh = jnp.dot(p, vh, preferred_element_type=jnp.float32) / l
            outs.append(oh)
        o = jnp.concatenate(outs, axis=1)
        acc_ref[...] = jnp.dot(o, wo_ref[...], preferred_element_type=jnp.float32)

        s0 = b2 * 128
        s1 = s0 + b1 * 64
        s2 = s1 + b0 * 32
        halving = [
            (0, rbuf0, 128, (1 - b2) * 128, s0),
            (1, rbuf1, 64, s0 + (1 - b1) * 64, s1),
            (2, rbuf2, 32, s1 + (1 - b0) * 32, s2),
        ]
        for r, rbuf, size, send_start, keep_start in halving:
            rdma = pltpu.make_async_remote_copy(
                src_ref=acc_ref.at[pl.ds(send_start, size), :],
                dst_ref=rbuf,
                send_sem=send_sems.at[r],
                recv_sem=recv_sems.at[r],
                device_id=(partners[r],),
                device_id_type=pl.DeviceIdType.MESH,
            )
            rdma.start()
            rdma.wait_recv()
            acc_ref[pl.ds(keep_start, size), :] += rbuf[...]
            rdma.wait_send()

        doubling = [
            (3, 2, 32, s2),
            (4, 1, 64, s1),
            (5, 0, 128, s0),
        ]
        for r, pi, size, own_start in doubling:
            rdma = pltpu.make_async_remote_copy(
                src_ref=acc_ref.at[pl.ds(own_start, size), :],
                dst_ref=acc_ref.at[pl.ds(own_start, size), :],
                send_sem=send_sems.at[r],
                recv_sem=recv_sems.at[r],
                device_id=(partners[pi],),
                device_id_type=pl.DeviceIdType.MESH,
            )
            rdma.start()
            rdma.wait_recv()
            rdma.wait_send()

        out_ref[0, :, :] = acc_ref[...]

    return pl.pallas_call(
        body,
        out_shape=jax.ShapeDtypeStruct((1, SQ, D), jnp.float32),
        in_specs=[pl.BlockSpec(memory_space=pltpu.VMEM)] * 5,
        out_specs=pl.BlockSpec(memory_space=pltpu.VMEM),
        scratch_shapes=[
            pltpu.VMEM((SQ, D), jnp.float32),
            pltpu.VMEM((128, D), jnp.float32),
            pltpu.VMEM((64, D), jnp.float32),
            pltpu.VMEM((32, D), jnp.float32),
            pltpu.SemaphoreType.DMA((6,)),
            pltpu.SemaphoreType.DMA((6,)),
        ],
        compiler_params=pltpu.CompilerParams(collective_id=0),
    )(x, Wq, Wo, Wk_s, Wv_s)


# device time: 44993 ns/iter; 2.3896x vs baseline; 1.0371x over previous
import jax
import jax.numpy as jnp
from jax import lax
from jax.experimental import pallas as pl
from jax.experimental.pallas import tpu as pltpu

N_DEV = 8
SQ = 256
D = 1024
DH = 128
HQ_PER = 8
KV_COLS = 256
SCALE = 0.08838834764831843


def kernel(x, Wq, Wo, Wk, Wv):
    i = lax.axis_index("i")
    Wk_s = lax.dynamic_slice(Wk, (0, i * KV_COLS), (D, KV_COLS))
    Wv_s = lax.dynamic_slice(Wv, (0, i * KV_COLS), (D, KV_COLS))

    def body(x_ref, wq_ref, wo_ref, wk_ref, wv_ref, out_ref,
             acc_ref, rbuf0, rbuf1, rbuf2, send_sems, recv_sems):
        my = lax.axis_index("i")
        b2 = (my >> 2) & 1
        b1 = (my >> 1) & 1
        b0 = my & 1
        partners = [my ^ 4, my ^ 3, my ^ 1]

        barrier = pltpu.get_barrier_semaphore()
        for p in partners:
            pl.semaphore_signal(
                barrier, inc=1,
                device_id=(p,), device_id_type=pl.DeviceIdType.MESH,
            )
        pl.semaphore_wait(barrier, 3)

        xv = x_ref[0, :, :]
        q = jnp.dot(xv, wq_ref[...], preferred_element_type=jnp.float32)
        k = jnp.dot(xv, wk_ref[...], preferred_element_type=jnp.float32)
        v = jnp.dot(xv, wv_ref[...], preferred_element_type=jnp.float32)

        outs = []
        for h in range(HQ_PER):
            qh = q[:, h * DH:(h + 1) * DH]
            g = h // 4
            kh = k[:, g * DH:(g + 1) * DH]
            vh = v[:, g * DH:(g + 1) * DH]
            s = lax.dot_general(
                qh, kh, (((1,), (1,)), ((), ())),
                preferred_element_type=jnp.float32,
            ) * SCALE
            m = jnp.max(s, axis=-1, keepdims=True)
            p = jnp.exp(s - m)
            l = jnp.sum(p, axis=-1, keepdims=True)
            oh = jnp.dot(p, vh, preferred_element_type=jnp.float32) / l
            outs.append(oh)
        o = jnp.concatenate(outs, axis=1)
        acc_ref[...] = jnp.dot(o, wo_ref[...], preferred_element_type=jnp.float32)

        s0 = b2 * 128
        s1 = s0 + b1 * 64
        s2 = s1 + b0 * 32
        halving = [
            (0, rbuf0, 128, (1 - b2) * 128, s0),
            (1, rbuf1, 64, s0 + (1 - b1) * 64, s1),
            (2, rbuf2, 32, s1 + (1 - b0) * 32, s2),
        ]
        for r, rbuf, size, send_start, keep_start in halving:
            rdma = pltpu.make_async_remote_copy(
                src_ref=acc_ref.at[pl.ds(send_start, size), :],
                dst_ref=rbuf,
                send_sem=send_sems.at[r],
                recv_sem=recv_sems.at[r],
                device_id=(partners[r],),
                device_id_type=pl.DeviceIdType.MESH,
            )
            rdma.start()
            rdma.wait_recv()
            acc_ref[pl.ds(keep_start, size), :] += rbuf[...]
            rdma.wait_send()

        doubling = [
            (3, 2, 32, s2),
            (4, 1, 64, s1),
            (5, 0, 128, s0),
        ]
        for r, pi, size, own_start in doubling:
            rdma = pltpu.make_async_remote_copy(
                src_ref=acc_ref.at[pl.ds(own_start, size), :],
                dst_ref=acc_ref.at[pl.ds(own_start, size), :],
                send_sem=send_sems.at[r],
                recv_sem=recv_sems.at[r],
                device_id=(partners[pi],),
                device_id_type=pl.DeviceIdType.MESH,
            )
            rdma.start()
            rdma.wait_recv()
            rdma.wait_send()

        out_ref[0, :, :] = acc_ref[...]

    return pl.pallas_call(
        body,
        out_shape=jax.ShapeDtypeStruct((1, SQ, D), jnp.float32),
        in_specs=[pl.BlockSpec(memory_space=pltpu.VMEM)] * 5,
        out_specs=pl.BlockSpec(memory_space=pltpu.VMEM),
        scratch_shapes=[
            pltpu.VMEM((SQ, D), jnp.float32),
            pltpu.VMEM((128, D), jnp.float32),
            pltpu.VMEM((64, D), jnp.float32),
            pltpu.VMEM((32, D), jnp.float32),
            pltpu.SemaphoreType.DMA((6,)),
            pltpu.SemaphoreType.DMA((6,)),
        ],
        compiler_params=pltpu.CompilerParams(collective_id=0),
    )(x, Wq, Wo, Wk_s, Wv_s)


# device time: 44537 ns/iter; 2.4141x vs baseline; 1.0102x over previous
import jax
import jax.numpy as jnp
from jax import lax
from jax.experimental import pallas as pl
from jax.experimental.pallas import tpu as pltpu

N_DEV = 8
SQ = 256
D = 1024
DH = 128
HQ_PER = 8
KV_COLS = 256
SCALE = 0.08838834764831843


def kernel(x, Wq, Wo, Wk, Wv):
    i = lax.axis_index("i")
    Wk_s = lax.dynamic_slice(Wk, (0, i * KV_COLS), (D, KV_COLS))
    Wv_s = lax.dynamic_slice(Wv, (0, i * KV_COLS), (D, KV_COLS))

    def body(x_ref, wq_ref, wo_ref, wk_ref, wv_ref, out_ref,
             rbuf0, rbuf1, rbuf2, send_sems, recv_sems):
        my = lax.axis_index("i")
        b2 = (my >> 2) & 1
        b1 = (my >> 1) & 1
        b0 = my & 1
        partners = [my ^ 4, my ^ 3, my ^ 1]
        acc = out_ref.at[0]

        barrier = pltpu.get_barrier_semaphore()
        for p in partners:
            pl.semaphore_signal(
                barrier, inc=1,
                device_id=(p,), device_id_type=pl.DeviceIdType.MESH,
            )
        pl.semaphore_wait(barrier, 3)

        xv = x_ref[0, :, :]
        q = jnp.dot(xv, wq_ref[...], preferred_element_type=jnp.float32)
        k = jnp.dot(xv, wk_ref[...], preferred_element_type=jnp.float32)
        v = jnp.dot(xv, wv_ref[...], preferred_element_type=jnp.float32)

        def attn_rows(r0_):
            outs = []
            for h in range(HQ_PER):
                qh = q[r0_:r0_ + 128, h * DH:(h + 1) * DH]
                g = h // 4
                kh = k[:, g * DH:(g + 1) * DH]
                vh = v[:, g * DH:(g + 1) * DH]
                s = lax.dot_general(
                    qh, kh, (((1,), (1,)), ((), ())),
                    preferred_element_type=jnp.float32,
                ) * SCALE
                m = jnp.max(s, axis=-1, keepdims=True)
                p = jnp.exp(s - m)
                l = jnp.sum(p, axis=-1, keepdims=True)
                outs.append(jnp.dot(p, vh, preferred_element_type=jnp.float32) / l)
            o = jnp.concatenate(outs, axis=1)
            acc[r0_:r0_ + 128, :] = jnp.dot(
                o, wo_ref[...], preferred_element_type=jnp.float32)

        s0 = b2 * 128
        s1 = s0 + b1 * 64
        s2 = s1 + b0 * 32
        send0 = (1 - b2) * 128

        pl.when(b2 == 1)(lambda: attn_rows(0))
        pl.when(b2 == 0)(lambda: attn_rows(128))
        rdma0 = pltpu.make_async_remote_copy(
            src_ref=acc.at[pl.ds(send0, 128), :],
            dst_ref=rbuf0,
            send_sem=send_sems.at[0],
            recv_sem=recv_sems.at[0],
            device_id=(partners[0],),
            device_id_type=pl.DeviceIdType.MESH,
        )
        rdma0.start()
        pl.when(b2 == 1)(lambda: attn_rows(128))
        pl.when(b2 == 0)(lambda: attn_rows(0))
        rdma0.wait_recv()
        acc[pl.ds(s0, 128), :] += rbuf0[...]
        rdma0.wait_send()

        for r, rbuf, size, send_start, keep_start in (
            (1, rbuf1, 64, s0 + (1 - b1) * 64, s1),
            (2, rbuf2, 32, s1 + (1 - b0) * 32, s2),
        ):
            rdma = pltpu.make_async_remote_copy(
                src_ref=acc.at[pl.ds(send_start, size), :],
                dst_ref=rbuf,
                send_sem=send_sems.at[r],
                recv_sem=recv_sems.at[r],
                device_id=(partners[r],),
                device_id_type=pl.DeviceIdType.MESH,
            )
            rdma.start()
            rdma.wait_recv()
            acc[pl.ds(keep_start, size), :] += rbuf[...]
            rdma.wait_send()

        for r, pi, size, own_start in (
            (3, 2, 32, s2),
            (4, 1, 64, s1),
            (5, 0, 128, s0),
        ):
            rdma = pltpu.make_async_remote_copy(
                src_ref=acc.at[pl.ds(own_start, size), :],
                dst_ref=acc.at[pl.ds(own_start, size), :],
                send_sem=send_sems.at[r],
                recv_sem=recv_sems.at[r],
                device_id=(partners[pi],),
                device_id_type=pl.DeviceIdType.MESH,
            )
            rdma.start()
            rdma.wait_recv()
            rdma.wait_send()

    return pl.pallas_call(
        body,
        out_shape=jax.ShapeDtypeStruct((1, SQ, D), jnp.float32),
        in_specs=[pl.BlockSpec(memory_space=pltpu.VMEM)] * 5,
        out_specs=pl.BlockSpec(memory_space=pltpu.VMEM),
        scratch_shapes=[
            pltpu.VMEM((128, D), jnp.float32),
            pltpu.VMEM((64, D), jnp.float32),
            pltpu.VMEM((32, D), jnp.float32),
            pltpu.SemaphoreType.DMA((6,)),
            pltpu.SemaphoreType.DMA((6,)),
        ],
        compiler_params=pltpu.CompilerParams(collective_id=0),
    )(x, Wq, Wo, Wk_s, Wv_s)


# device time: 33209 ns/iter; 3.2376x vs baseline; 1.3411x over previous
import jax
import jax.numpy as jnp
from jax import lax
from jax.experimental import pallas as pl
from jax.experimental.pallas import tpu as pltpu

N_DEV = 8
SQ = 256
D = 1024
DH = 128
HQ_PER = 8
KV_COLS = 256
CH = SQ // N_DEV
SCALE = 0.08838834764831843


def kernel(x, Wq, Wo, Wk, Wv):
    i = lax.axis_index("i")
    Wk_s = lax.dynamic_slice(Wk, (0, i * KV_COLS), (D, KV_COLS))
    Wv_s = lax.dynamic_slice(Wv, (0, i * KV_COLS), (D, KV_COLS))

    def body(x_ref, wq_ref, wo_ref, wk_ref, wv_ref, out_ref,
             pbuf, scatter_buf, ssend_sems, srecv_sems, bsend_sems,
             brecv_sems):
        my = lax.axis_index("i")
        acc = out_ref.at[0]
        my_rows = pl.ds(CH * my, CH)

        barrier = pltpu.get_barrier_semaphore()
        for p in range(N_DEV):
            pl.when(my != p)(lambda p=p: pl.semaphore_signal(
                barrier, inc=1,
                device_id=(p,), device_id_type=pl.DeviceIdType.MESH,
            ))
        pl.semaphore_wait(barrier, N_DEV - 1)

        xv = x_ref[0, :, :]
        q = jnp.dot(xv, wq_ref[...], preferred_element_type=jnp.float32)
        k = jnp.dot(xv, wk_ref[...], preferred_element_type=jnp.float32)
        v = jnp.dot(xv, wv_ref[...], preferred_element_type=jnp.float32)

        outs = []
        for h in range(HQ_PER):
            qh = q[:, h * DH:(h + 1) * DH]
            g = h // 4
            kh = k[:, g * DH:(g + 1) * DH]
            vh = v[:, g * DH:(g + 1) * DH]
            s = lax.dot_general(
                qh, kh, (((1,), (1,)), ((), ())),
                preferred_element_type=jnp.float32,
            ) * SCALE
            m = jnp.max(s, axis=-1, keepdims=True)
            p = jnp.exp(s - m)
            l = jnp.sum(p, axis=-1, keepdims=True)
            outs.append(jnp.dot(p, vh, preferred_element_type=jnp.float32) / l)
        o = jnp.concatenate(outs, axis=1)
        pbuf[...] = jnp.dot(o, wo_ref[...], preferred_element_type=jnp.float32)

        def p1_send(c):
            rdma = pltpu.make_async_remote_copy(
                src_ref=pbuf.at[pl.ds(CH * c, CH), :],
                dst_ref=scatter_buf.at[my],
                send_sem=ssend_sems.at[c],
                recv_sem=srecv_sems.at[my],
                device_id=(c,),
                device_id_type=pl.DeviceIdType.MESH,
            )
            rdma.start()
        for c in range(N_DEV):
            pl.when(my != c)(lambda c=c: p1_send(c))

        scatter_buf[my] = pbuf[my_rows, :]
        for j in range(N_DEV):
            def p1_wait(j=j):
                recv = pltpu.make_async_remote_copy(
                    src_ref=scatter_buf.at[j],
                    dst_ref=scatter_buf.at[j],
                    send_sem=ssend_sems.at[j],
                    recv_sem=srecv_sems.at[j],
                    device_id=(j,),
                    device_id_type=pl.DeviceIdType.MESH,
                )
                recv.wait_recv()
            pl.when(my != j)(p1_wait)

        red = scatter_buf[0]
        for j in range(1, N_DEV):
            red = red + scatter_buf[j]
        acc[my_rows, :] = red

        def p2_send(c):
            rdma = pltpu.make_async_remote_copy(
                src_ref=acc.at[my_rows, :],
                dst_ref=acc.at[my_rows, :],
                send_sem=bsend_sems.at[c],
                recv_sem=brecv_sems.at[my],
                device_id=(c,),
                device_id_type=pl.DeviceIdType.MESH,
            )
            rdma.start()
        for c in range(N_DEV):
            pl.when(my != c)(lambda c=c: p2_send(c))

        for j in range(N_DEV):
            def p2_wait(j=j):
                recv = pltpu.make_async_remote_copy(
                    src_ref=acc.at[pl.ds(CH * j, CH), :],
                    dst_ref=acc.at[pl.ds(CH * j, CH), :],
                    send_sem=bsend_sems.at[j],
                    recv_sem=brecv_sems.at[j],
                    device_id=(j,),
                    device_id_type=pl.DeviceIdType.MESH,
                )
                recv.wait_recv()
            pl.when(my != j)(p2_wait)

        for c in range(N_DEV):
            def drain(c=c):
                s1 = pltpu.make_async_remote_copy(
                    src_ref=pbuf.at[pl.ds(CH * c, CH), :],
                    dst_ref=scatter_buf.at[my],
                    send_sem=ssend_sems.at[c],
                    recv_sem=srecv_sems.at[my],
                    device_id=(c,),
                    device_id_type=pl.DeviceIdType.MESH,
                )
                s1.wait_send()
                s2 = pltpu.make_async_remote_copy(
                    src_ref=acc.at[my_rows, :],
                    dst_ref=acc.at[my_rows, :],
                    send_sem=bsend_sems.at[c],
                    recv_sem=brecv_sems.at[my],
                    device_id=(c,),
                    device_id_type=pl.DeviceIdType.MESH,
                )
                s2.wait_send()
            pl.when(my != c)(drain)

    return pl.pallas_call(
        body,
        out_shape=jax.ShapeDtypeStruct((1, SQ, D), jnp.float32),
        in_specs=[pl.BlockSpec(memory_space=pltpu.VMEM)] * 5,
        out_specs=pl.BlockSpec(memory_space=pltpu.VMEM),
        scratch_shapes=[
            pltpu.VMEM((SQ, D), jnp.float32),
            pltpu.VMEM((N_DEV, CH, D), jnp.float32),
            pltpu.SemaphoreType.DMA((N_DEV,)),
            pltpu.SemaphoreType.DMA((N_DEV,)),
            pltpu.SemaphoreType.DMA((N_DEV,)),
            pltpu.SemaphoreType.DMA((N_DEV,)),
        ],
        compiler_params=pltpu.CompilerParams(collective_id=0),
    )(x, Wq, Wo, Wk_s, Wv_s)


# device time: 27818 ns/iter; 3.8650x vs baseline; 1.1938x over previous
import jax
import jax.numpy as jnp
from jax import lax
from jax.experimental import pallas as pl
from jax.experimental.pallas import tpu as pltpu

N_DEV = 8
SQ = 256
D = 1024
DH = 128
HQ_PER = 8
KV_COLS = 256
CH = SQ // N_DEV
SCALE = 0.08838834764831843


def kernel(x, Wq, Wo, Wk, Wv):
    i = lax.axis_index("i")
    Wk_s = lax.dynamic_slice(Wk, (0, i * KV_COLS), (D, KV_COLS))
    Wv_s = lax.dynamic_slice(Wv, (0, i * KV_COLS), (D, KV_COLS))

    def body(x_ref, wq_ref, wo_ref, wk_ref, wv_ref, out_ref,
             pbuf32, pbuf, scatter_buf, bbuf, bcast_buf,
             ssend_sems, srecv_sems, bsend_sems, brecv_sems):
        my = lax.axis_index("i")
        acc = out_ref.at[0]
        my_rows = pl.ds(CH * my, CH)

        barrier = pltpu.get_barrier_semaphore()
        for p in range(N_DEV):
            pl.when(my != p)(lambda p=p: pl.semaphore_signal(
                barrier, inc=1,
                device_id=(p,), device_id_type=pl.DeviceIdType.MESH,
            ))
        pl.semaphore_wait(barrier, N_DEV - 1)

        xv = x_ref[0, :, :]
        q = jnp.dot(xv, wq_ref[...], preferred_element_type=jnp.float32)
        k = jnp.dot(xv, wk_ref[...], preferred_element_type=jnp.float32)
        v = jnp.dot(xv, wv_ref[...], preferred_element_type=jnp.float32)

        outs = []
        for h in range(HQ_PER):
            qh = q[:, h * DH:(h + 1) * DH]
            g = h // 4
            kh = k[:, g * DH:(g + 1) * DH]
            vh = v[:, g * DH:(g + 1) * DH]
            s = lax.dot_general(
                qh, kh, (((1,), (1,)), ((), ())),
                preferred_element_type=jnp.float32,
            ) * SCALE
            m = jnp.max(s, axis=-1, keepdims=True)
            p = jnp.exp(s - m)
            l = jnp.sum(p, axis=-1, keepdims=True)
            outs.append(jnp.dot(p, vh, preferred_element_type=jnp.float32) / l)
        o = jnp.concatenate(outs, axis=1)

        for c in range(N_DEV):
            rows = slice(CH * c, CH * (c + 1))
            chunk = jnp.dot(o[rows, :], wo_ref[...],
                            preferred_element_type=jnp.float32)
            pbuf32[rows, :] = chunk
            pbuf[rows, :] = chunk.astype(jnp.bfloat16)

            def p1_send(c=c):
                rdma = pltpu.make_async_remote_copy(
                    src_ref=pbuf.at[pl.ds(CH * c, CH), :],
                    dst_ref=scatter_buf.at[my],
                    send_sem=ssend_sems.at[c],
                    recv_sem=srecv_sems.at[my],
                    device_id=(c,),
                    device_id_type=pl.DeviceIdType.MESH,
                )
                rdma.start()
            pl.when(my != c)(p1_send)

        scatter_buf[my] = jnp.zeros((CH, D), jnp.bfloat16)
        for j in range(N_DEV):
            def p1_wait(j=j):
                recv = pltpu.make_async_remote_copy(
                    src_ref=scatter_buf.at[j],
                    dst_ref=scatter_buf.at[j],
                    send_sem=ssend_sems.at[j],
                    recv_sem=srecv_sems.at[j],
                    device_id=(j,),
                    device_id_type=pl.DeviceIdType.MESH,
                )
                recv.wait_recv()
            pl.when(my != j)(p1_wait)

        red = pbuf32[my_rows, :]
        for j in range(N_DEV):
            red = red + scatter_buf[j].astype(jnp.float32)
        acc[my_rows, :] = red
        bbuf[...] = red.astype(jnp.bfloat16)

        for c in range(N_DEV):
            def p2_send(c=c):
                rdma = pltpu.make_async_remote_copy(
                    src_ref=bbuf,
                    dst_ref=bcast_buf.at[my],
                    send_sem=bsend_sems.at[c],
                    recv_sem=brecv_sems.at[my],
                    device_id=(c,),
                    device_id_type=pl.DeviceIdType.MESH,
                )
                rdma.start()
            pl.when(my != c)(p2_send)

        for j in range(N_DEV):
            def p2_take(j=j):
                recv = pltpu.make_async_remote_copy(
                    src_ref=bcast_buf.at[j],
                    dst_ref=bcast_buf.at[j],
                    send_sem=bsend_sems.at[j],
                    recv_sem=brecv_sems.at[j],
                    device_id=(j,),
                    device_id_type=pl.DeviceIdType.MESH,
                )
                recv.wait_recv()
                acc[CH * j:CH * (j + 1), :] = (
                    bcast_buf[j].astype(jnp.float32))
            pl.when(my != j)(p2_take)

        for c in range(N_DEV):
            def drain(c=c):
                s1 = pltpu.make_async_remote_copy(
                    src_ref=pbuf.at[pl.ds(CH * c, CH), :],
                    dst_ref=scatter_buf.at[my],
                    send_sem=ssend_sems.at[c],
                    recv_sem=srecv_sems.at[my],
                    device_id=(c,),
                    device_id_type=pl.DeviceIdType.MESH,
                )
                s1.wait_send()
                s2 = pltpu.make_async_remote_copy(
                    src_ref=bbuf,
                    dst_ref=bcast_buf.at[my],
                    send_sem=bsend_sems.at[c],
                    recv_sem=brecv_sems.at[my],
                    device_id=(c,),
                    device_id_type=pl.DeviceIdType.MESH,
                )
                s2.wait_send()
            pl.when(my != c)(drain)

    return pl.pallas_call(
        body,
        out_shape=jax.ShapeDtypeStruct((1, SQ, D), jnp.float32),
        in_specs=[pl.BlockSpec(memory_space=pltpu.VMEM)] * 5,
        out_specs=pl.BlockSpec(memory_space=pltpu.VMEM),
        scratch_shapes=[
            pltpu.VMEM((SQ, D), jnp.float32),
            pltpu.VMEM((SQ, D), jnp.bfloat16),
            pltpu.VMEM((N_DEV, CH, D), jnp.bfloat16),
            pltpu.VMEM((CH, D), jnp.bfloat16),
            pltpu.VMEM((N_DEV, CH, D), jnp.bfloat16),
            pltpu.SemaphoreType.DMA((N_DEV,)),
            pltpu.SemaphoreType.DMA((N_DEV,)),
            pltpu.SemaphoreType.DMA((N_DEV,)),
            pltpu.SemaphoreType.DMA((N_DEV,)),
        ],
        compiler_params=pltpu.CompilerParams(collective_id=0),
    )(x, Wq, Wo, Wk_s, Wv_s)


# device time: 25162 ns/iter; 4.2730x vs baseline; 1.1056x over previous
import jax
import jax.numpy as jnp
from jax import lax
from jax.experimental import pallas as pl
from jax.experimental.pallas import tpu as pltpu

N_DEV = 8
SQ = 256
D = 1024
DH = 128
HQ_PER = 8
KV_COLS = 256
CH = SQ // N_DEV
SCALE = 0.08838834764831843


def kernel(x, Wq, Wo, Wk, Wv):
    i = lax.axis_index("i")
    Wk_s = lax.dynamic_slice(Wk, (0, i * KV_COLS), (D, KV_COLS))
    Wv_s = lax.dynamic_slice(Wv, (0, i * KV_COLS), (D, KV_COLS))

    def body(x_ref, wq_ref, wo_ref, wk_ref, wv_ref, out_ref,
             pbuf32, pbuf, scatter_buf, bbuf, bcast_buf,
             ssend_sems, srecv_sems, bsend_sems, brecv_sems):
        my = lax.axis_index("i")
        acc = out_ref.at[0]
        my_rows = pl.ds(CH * my, CH)

        barrier = pltpu.get_barrier_semaphore()
        for p in range(N_DEV):
            pl.when(my != p)(lambda p=p: pl.semaphore_signal(
                barrier, inc=1,
                device_id=(p,), device_id_type=pl.DeviceIdType.MESH,
            ))

        xv = x_ref[0, :, :].astype(jnp.bfloat16)
        q = jnp.dot(xv, wq_ref[...].astype(jnp.bfloat16),
                    preferred_element_type=jnp.float32)
        k = jnp.dot(xv, wk_ref[...].astype(jnp.bfloat16),
                    preferred_element_type=jnp.float32)
        v = jnp.dot(xv, wv_ref[...].astype(jnp.bfloat16),
                    preferred_element_type=jnp.float32)

        outs = []
        for h in range(HQ_PER):
            qh = q[:, h * DH:(h + 1) * DH].astype(jnp.bfloat16)
            g = h // 4
            kh = k[:, g * DH:(g + 1) * DH].astype(jnp.bfloat16)
            vh = v[:, g * DH:(g + 1) * DH].astype(jnp.bfloat16)
            s = lax.dot_general(
                qh, kh, (((1,), (1,)), ((), ())),
                preferred_element_type=jnp.float32,
            ) * SCALE
            m = jnp.max(s, axis=-1, keepdims=True)
            p = jnp.exp(s - m)
            l = jnp.sum(p, axis=-1, keepdims=True)
            ph = p.astype(jnp.bfloat16)
            outs.append(jnp.dot(ph, vh, preferred_element_type=jnp.float32) / l)
        o = jnp.concatenate(outs, axis=1).astype(jnp.bfloat16)
        wo_bf = wo_ref[...].astype(jnp.bfloat16)

        pl.semaphore_wait(barrier, N_DEV - 1)

        for c in range(N_DEV):
            rows = slice(CH * c, CH * (c + 1))
            chunk = jnp.dot(o[rows, :], wo_bf,
                            preferred_element_type=jnp.float32)
            pbuf32[rows, :] = chunk
            pbuf[rows, :] = chunk.astype(jnp.bfloat16)

            def p1_send(c=c):
                rdma = pltpu.make_async_remote_copy(
                    src_ref=pbuf.at[pl.ds(CH * c, CH), :],
                    dst_ref=scatter_buf.at[my],
                    send_sem=ssend_sems.at[c],
                    recv_sem=srecv_sems.at[my],
                    device_id=(c,),
                    device_id_type=pl.DeviceIdType.MESH,
                )
                rdma.start()
            pl.when(my != c)(p1_send)

        scatter_buf[my] = jnp.zeros((CH, D), jnp.bfloat16)
        for j in range(N_DEV):
            def p1_wait(j=j):
                recv = pltpu.make_async_remote_copy(
                    src_ref=scatter_buf.at[j],
                    dst_ref=scatter_buf.at[j],
                    send_sem=ssend_sems.at[j],
                    recv_sem=srecv_sems.at[j],
                    device_id=(j,),
                    device_id_type=pl.DeviceIdType.MESH,
                )
                recv.wait_recv()
            pl.when(my != j)(p1_wait)

        red = pbuf32[my_rows, :]
        for j in range(N_DEV):
            red = red + scatter_buf[j].astype(jnp.float32)
        acc[my_rows, :] = red
        bbuf[...] = red.astype(jnp.bfloat16)

        for c in range(N_DEV):
            def p2_send(c=c):
                rdma = pltpu.make_async_remote_copy(
                    src_ref=bbuf,
                    dst_ref=bcast_buf.at[my],
                    send_sem=bsend_sems.at[c],
                    recv_sem=brecv_sems.at[my],
                    device_id=(c,),
                    device_id_type=pl.DeviceIdType.MESH,
                )
                rdma.start()
            pl.when(my != c)(p2_send)

        for j in range(N_DEV):
            def p2_take(j=j):
                recv = pltpu.make_async_remote_copy(
                    src_ref=bcast_buf.at[j],
                    dst_ref=bcast_buf.at[j],
                    send_sem=bsend_sems.at[j],
                    recv_sem=brecv_sems.at[j],
                    device_id=(j,),
                    device_id_type=pl.DeviceIdType.MESH,
                )
                recv.wait_recv()
                acc[CH * j:CH * (j + 1), :] = (
                    bcast_buf[j].astype(jnp.float32))
            pl.when(my != j)(p2_take)

        for c in range(N_DEV):
            def drain(c=c):
                s1 = pltpu.make_async_remote_copy(
                    src_ref=pbuf.at[pl.ds(CH * c, CH), :],
                    dst_ref=scatter_buf.at[my],
                    send_sem=ssend_sems.at[c],
                    recv_sem=srecv_sems.at[my],
                    device_id=(c,),
                    device_id_type=pl.DeviceIdType.MESH,
                )
                s1.wait_send()
                s2 = pltpu.make_async_remote_copy(
                    src_ref=bbuf,
                    dst_ref=bcast_buf.at[my],
                    send_sem=bsend_sems.at[c],
                    recv_sem=brecv_sems.at[my],
                    device_id=(c,),
                    device_id_type=pl.DeviceIdType.MESH,
                )
                s2.wait_send()
            pl.when(my != c)(drain)

    return pl.pallas_call(
        body,
        out_shape=jax.ShapeDtypeStruct((1, SQ, D), jnp.float32),
        in_specs=[pl.BlockSpec(memory_space=pltpu.VMEM)] * 5,
        out_specs=pl.BlockSpec(memory_space=pltpu.VMEM),
        scratch_shapes=[
            pltpu.VMEM((SQ, D), jnp.float32),
            pltpu.VMEM((SQ, D), jnp.bfloat16),
            pltpu.VMEM((N_DEV, CH, D), jnp.bfloat16),
            pltpu.VMEM((CH, D), jnp.bfloat16),
            pltpu.VMEM((N_DEV, CH, D), jnp.bfloat16),
            pltpu.SemaphoreType.DMA((N_DEV,)),
            pltpu.SemaphoreType.DMA((N_DEV,)),
            pltpu.SemaphoreType.DMA((N_DEV,)),
            pltpu.SemaphoreType.DMA((N_DEV,)),
        ],
        compiler_params=pltpu.CompilerParams(collective_id=0),
    )(x, Wq, Wo, Wk_s, Wv_s)
